# baseline (speedup 1.0000x reference)
import jax
import jax.numpy as jnp
from jax.experimental import pallas as pl
from jax.experimental.pallas import tpu as pltpu

N = 1024
NB = 1024
EMBED = 128
RNN = 256
B = 32
T = 50
ALPHA = 0.5

_RB = 128
_NI = N // _RB
_BB = 8
_XB = _BB * T


def _graph_body(F_ref, A_ref, out_ref, hb_s, h1z_s, deg_s, dinv_s):
    c = pl.program_id(0)
    p = pl.program_id(1)
    i = pl.program_id(2)
    rows = pl.ds(i * _RB, _RB)

    def combine(widx):
        acc = F_ref[widx, c, 0] * A_ref[0]
        for e in range(1, 4):
            acc = acc + F_ref[widx, c, e] * A_ref[e]
        return acc.astype(jnp.bfloat16)

    @pl.when(jnp.logical_and(p == 0, i == 0))
    def _():
        hb_s[...] = combine(1)
        deg_s[...] = jnp.zeros_like(deg_s)

    @pl.when(p == 0)
    def _():
        ha_i = F_ref[0, c, 0] * A_ref[0, rows, :]
        for e in range(1, 4):
            ha_i = ha_i + F_ref[0, c, e] * A_ref[e, rows, :]
        h1 = jnp.dot(ha_i.astype(jnp.bfloat16), hb_s[...],
                     preferred_element_type=jnp.float32)
        col = jax.lax.broadcasted_iota(jnp.int32, (_RB, N), 1)
        row = jax.lax.broadcasted_iota(jnp.int32, (_RB, N), 0) + i * _RB
        h1 = jnp.where(col == row, 0.0, h1)
        deg_s[...] += jnp.sum(h1, axis=0, keepdims=True)
        h1z_s[rows, :] = h1.astype(jnp.bfloat16)

    @pl.when(jnp.logical_and(p == 1, i == 0))
    def _():
        hb_s[...] = combine(2)
        deg = deg_s[...]
        dinv_s[...] = jnp.where(deg > 0, 1.0 / deg, 0.0)

    @pl.when(p == 1)
    def _():
        h1z = (h1z_s[rows, :] * dinv_s[...]).astype(jnp.bfloat16)
        out_ref[...] = jnp.dot(h1z, hb_s[...],
                               preferred_element_type=jnp.float32
                               ).astype(jnp.bfloat16)[None]


def _basket_body(x_ref, h2_ref, ib_ref, linW_ref, linb_ref, projW_ref,
                 projb_ref, out_ref):
    xb = x_ref[...]
    xb16 = xb.astype(jnp.bfloat16)
    ib = jnp.maximum(ib_ref[...], 0.0)
    xd = xb * ib
    t0 = jnp.dot(xb16, h2_ref[0], preferred_element_type=jnp.float32)
    e0 = jnp.maximum(xd, 0.0) + jnp.maximum(t0, 0.0)
    t1 = jnp.dot(xb16, h2_ref[1], preferred_element_type=jnp.float32)
    e1 = jnp.maximum(xd + jnp.maximum(t1, 0.0), 0.0)
    dn = (((1,), (1,)), ((), ()))
    eb0 = jax.lax.dot_general(e0.astype(jnp.bfloat16), linW_ref[0], dn,
                              preferred_element_type=jnp.float32) + linb_ref[0]
    eb1 = jax.lax.dot_general(e1.astype(jnp.bfloat16), linW_ref[1], dn,
                              preferred_element_type=jnp.float32) + linb_ref[1]
    comb = (jax.lax.dot_general(eb0.astype(jnp.bfloat16),
                                projW_ref[:, 0:EMBED], dn,
                                preferred_element_type=jnp.float32)
            + jax.lax.dot_general(eb1.astype(jnp.bfloat16),
                                  projW_ref[:, EMBED:2 * EMBED], dn,
                                  preferred_element_type=jnp.float32)
            + projb_ref[...])
    for b in range(_BB):
        out_ref[:, b, :] = comb[b * T:(b + 1) * T, :]


def _lstm_body(xs_ref, wih_ref, whh_ref, bih_ref, bhh_ref, h0_ref, c0_ref,
               sl_ref, h2i_ref, ib_ref, out_ref, xg_s):
    dn = (((1,), (1,)), ((), ()))
    xs_flat = xs_ref[...].reshape(T * B, EMBED)
    xg_s[...] = (jax.lax.dot_general(xs_flat, wih_ref[...], dn,
                                     preferred_element_type=jnp.float32)
                 + bih_ref[...] + bhh_ref[...])
    tgt = sl_ref[...] - 1

    def step(t, carry):
        h, c, acc = carry
        xt = xg_s[pl.ds(t * B, B), :]
        gates = xt + jax.lax.dot_general(h, whh_ref[...], dn,
                                         preferred_element_type=jnp.float32)
        i_ = jax.nn.sigmoid(gates[:, 0:RNN])
        f_ = jax.nn.sigmoid(gates[:, RNN:2 * RNN])
        g_ = jnp.tanh(gates[:, 2 * RNN:3 * RNN])
        o_ = jax.nn.sigmoid(gates[:, 3 * RNN:4 * RNN])
        c = f_ * c + i_ * g_
        h = o_ * jnp.tanh(c)
        acc = jnp.where(tgt == t, h, acc)
        return h, c, acc

    h0 = h0_ref[...]
    _, _, actual = jax.lax.fori_loop(0, T, step, (h0, c0_ref[...],
                                                  jnp.zeros_like(h0)))
    scores = jax.lax.dot_general(actual, h2i_ref[...], dn,
                                 preferred_element_type=jnp.float32)
    probs = jax.nn.sigmoid(scores)
    ib = jnp.maximum(ib_ref[...], 0.0)
    out_ref[...] = (1.0 - ALPHA) * probs + ALPHA * (probs * ib)


def kernel(A, seqs, seq_len, h0, c0, Wc1, Wc2, Wc3, I_B, lin_W, lin_b,
           proj_W, proj_b, W_ih, W_hh, b_ih, b_hh, h2i_W):
    F = jax.nn.softmax(jnp.stack([Wc1, Wc2, Wc3]), axis=2)

    H2 = pl.pallas_call(
        _graph_body,
        grid=(2, 2, _NI),
        in_specs=[
            pl.BlockSpec(memory_space=pltpu.SMEM),
            pl.BlockSpec((4, N, N), lambda c, p, i: (0, 0, 0)),
        ],
        out_specs=pl.BlockSpec((1, _RB, N), lambda c, p, i: (c, p * i, 0)),
        out_shape=jax.ShapeDtypeStruct((2, N, N), jnp.bfloat16),
        scratch_shapes=[
            pltpu.VMEM((N, N), jnp.bfloat16),
            pltpu.VMEM((N, N), jnp.bfloat16),
            pltpu.VMEM((1, N), jnp.float32),
            pltpu.VMEM((1, N), jnp.float32),
        ],
        compiler_params=pltpu.CompilerParams(
            dimension_semantics=("arbitrary", "arbitrary", "arbitrary")),
    )(F, A)

    x = seqs.reshape(B * T, NB)
    ib2 = I_B.reshape(1, NB)
    xs_t = pl.pallas_call(
        _basket_body,
        grid=(B // _BB,),
        in_specs=[
            pl.BlockSpec((_XB, NB), lambda i: (i, 0)),
            pl.BlockSpec((2, N, N), lambda i: (0, 0, 0)),
            pl.BlockSpec((1, NB), lambda i: (0, 0)),
            pl.BlockSpec((2, EMBED, NB), lambda i: (0, 0, 0)),
            pl.BlockSpec((2, 1, EMBED), lambda i: (0, 0, 0)),
            pl.BlockSpec((EMBED, 2 * EMBED), lambda i: (0, 0)),
            pl.BlockSpec((1, EMBED), lambda i: (0, 0)),
        ],
        out_specs=pl.BlockSpec((T, _BB, EMBED), lambda i: (0, i, 0)),
        out_shape=jax.ShapeDtypeStruct((T, B, EMBED), jnp.float32),
        compiler_params=pltpu.CompilerParams(
            dimension_semantics=("arbitrary",)),
    )(x, H2, ib2, lin_W.astype(jnp.bfloat16),
      lin_b.reshape(2, 1, EMBED), proj_W.astype(jnp.bfloat16),
      proj_b.reshape(1, EMBED))

    out = pl.pallas_call(
        _lstm_body,
        in_specs=[
            pl.BlockSpec((T, B, EMBED), lambda: (0, 0, 0)),
            pl.BlockSpec((4 * RNN, EMBED), lambda: (0, 0)),
            pl.BlockSpec((4 * RNN, RNN), lambda: (0, 0)),
            pl.BlockSpec((1, 4 * RNN), lambda: (0, 0)),
            pl.BlockSpec((1, 4 * RNN), lambda: (0, 0)),
            pl.BlockSpec((B, RNN), lambda: (0, 0)),
            pl.BlockSpec((B, RNN), lambda: (0, 0)),
            pl.BlockSpec((B, 1), lambda: (0, 0)),
            pl.BlockSpec((NB, RNN), lambda: (0, 0)),
            pl.BlockSpec((1, NB), lambda: (0, 0)),
        ],
        out_specs=pl.BlockSpec((B, NB), lambda: (0, 0)),
        out_shape=jax.ShapeDtypeStruct((B, NB), jnp.float32),
        scratch_shapes=[pltpu.VMEM((T * B, 4 * RNN), jnp.float32)],
    )(xs_t, W_ih, W_hh, b_ih.reshape(1, 4 * RNN), b_hh.reshape(1, 4 * RNN),
      h0.reshape(B, RNN), c0.reshape(B, RNN),
      seq_len.astype(jnp.int32).reshape(B, 1), h2i_W, ib2)

    return out


# speedup vs baseline: 1.0108x; 1.0108x over previous
import jax
import jax.numpy as jnp
from jax.experimental import pallas as pl
from jax.experimental.pallas import tpu as pltpu

N = 1024
NB = 1024
EMBED = 128
RNN = 256
B = 32
T = 50
ALPHA = 0.5

_RB = 128
_NI = N // _RB
_BB = 8
_XB = _BB * T


def _graph_body(F_ref, A_ref, out_ref, hb_s, h1z_s, deg_s, dinv_s):
    c = pl.program_id(0)
    p = pl.program_id(1)
    i = pl.program_id(2)
    rows = pl.ds(i * _RB, _RB)

    def combine(widx):
        acc = F_ref[widx, c, 0] * A_ref[0]
        for e in range(1, 4):
            acc = acc + F_ref[widx, c, e] * A_ref[e]
        return acc.astype(jnp.bfloat16)

    @pl.when(jnp.logical_and(p == 0, i == 0))
    def _():
        hb_s[...] = combine(1)
        deg_s[...] = jnp.zeros_like(deg_s)

    @pl.when(p == 0)
    def _():
        ha_i = F_ref[0, c, 0] * A_ref[0, rows, :]
        for e in range(1, 4):
            ha_i = ha_i + F_ref[0, c, e] * A_ref[e, rows, :]
        h1 = jnp.dot(ha_i.astype(jnp.bfloat16), hb_s[...],
                     preferred_element_type=jnp.float32)
        col = jax.lax.broadcasted_iota(jnp.int32, (_RB, N), 1)
        row = jax.lax.broadcasted_iota(jnp.int32, (_RB, N), 0) + i * _RB
        h1 = jnp.where(col == row, 0.0, h1)
        deg_s[...] += jnp.sum(h1, axis=0, keepdims=True)
        h1z_s[rows, :] = h1.astype(jnp.bfloat16)

    @pl.when(jnp.logical_and(p == 1, i == 0))
    def _():
        hb_s[...] = combine(2)
        deg = deg_s[...]
        dinv_s[...] = jnp.where(deg > 0, 1.0 / deg, 0.0)

    @pl.when(p == 1)
    def _():
        h1z = (h1z_s[rows, :] * dinv_s[...]).astype(jnp.bfloat16)
        out_ref[...] = jnp.dot(h1z, hb_s[...],
                               preferred_element_type=jnp.float32
                               ).astype(jnp.bfloat16)[None]


def _basket_body(x_ref, h2_ref, ib_ref, linW_ref, linb_ref, projW_ref,
                 projb_ref, out_ref):
    xb = x_ref[...]
    xb16 = xb.astype(jnp.bfloat16)
    ib = jnp.maximum(ib_ref[...], 0.0)
    xd = xb * ib
    t0 = jnp.dot(xb16, h2_ref[0], preferred_element_type=jnp.float32)
    e0 = jnp.maximum(xd, 0.0) + jnp.maximum(t0, 0.0)
    t1 = jnp.dot(xb16, h2_ref[1], preferred_element_type=jnp.float32)
    e1 = jnp.maximum(xd + jnp.maximum(t1, 0.0), 0.0)
    dn = (((1,), (1,)), ((), ()))
    eb0 = jax.lax.dot_general(e0.astype(jnp.bfloat16),
                              linW_ref[0].astype(jnp.bfloat16), dn,
                              preferred_element_type=jnp.float32) + linb_ref[0]
    eb1 = jax.lax.dot_general(e1.astype(jnp.bfloat16),
                              linW_ref[1].astype(jnp.bfloat16), dn,
                              preferred_element_type=jnp.float32) + linb_ref[1]
    comb = (jax.lax.dot_general(eb0.astype(jnp.bfloat16),
                                projW_ref[:, 0:EMBED].astype(jnp.bfloat16), dn,
                                preferred_element_type=jnp.float32)
            + jax.lax.dot_general(eb1.astype(jnp.bfloat16),
                                  projW_ref[:, EMBED:2 * EMBED].astype(
                                      jnp.bfloat16), dn,
                                  preferred_element_type=jnp.float32)
            + projb_ref[...])
    for b in range(_BB):
        out_ref[:, b, :] = comb[b * T:(b + 1) * T, :]


def _lstm_body(xs_ref, wih_ref, whh_ref, bih_ref, bhh_ref, h0_ref, c0_ref,
               sl_ref, h2i_ref, ib_ref, out_ref, xg_s):
    dn = (((1,), (1,)), ((), ()))
    xs_flat = xs_ref[...].reshape(T * B, EMBED)
    xg_s[...] = (jax.lax.dot_general(xs_flat, wih_ref[...], dn,
                                     preferred_element_type=jnp.float32)
                 + bih_ref[...] + bhh_ref[...])
    tgt = sl_ref[...] - 1

    def step(t, carry):
        h, c, acc = carry
        xt = xg_s[pl.ds(t * B, B), :]
        gates = xt + jax.lax.dot_general(h, whh_ref[...], dn,
                                         preferred_element_type=jnp.float32)
        i_ = jax.nn.sigmoid(gates[:, 0:RNN])
        f_ = jax.nn.sigmoid(gates[:, RNN:2 * RNN])
        g_ = jnp.tanh(gates[:, 2 * RNN:3 * RNN])
        o_ = jax.nn.sigmoid(gates[:, 3 * RNN:4 * RNN])
        c = f_ * c + i_ * g_
        h = o_ * jnp.tanh(c)
        acc = jnp.where(tgt == t, h, acc)
        return h, c, acc

    h0 = h0_ref[...]
    _, _, actual = jax.lax.fori_loop(0, T, step, (h0, c0_ref[...],
                                                  jnp.zeros_like(h0)))
    scores = jax.lax.dot_general(actual, h2i_ref[...], dn,
                                 preferred_element_type=jnp.float32)
    probs = jax.nn.sigmoid(scores)
    ib = jnp.maximum(ib_ref[...], 0.0)
    out_ref[...] = (1.0 - ALPHA) * probs + ALPHA * (probs * ib)


def kernel(A, seqs, seq_len, h0, c0, Wc1, Wc2, Wc3, I_B, lin_W, lin_b,
           proj_W, proj_b, W_ih, W_hh, b_ih, b_hh, h2i_W):
    F = jax.nn.softmax(jnp.stack([Wc1, Wc2, Wc3]), axis=2)

    H2 = pl.pallas_call(
        _graph_body,
        grid=(2, 2, _NI),
        in_specs=[
            pl.BlockSpec(memory_space=pltpu.SMEM),
            pl.BlockSpec((4, N, N), lambda c, p, i: (0, 0, 0)),
        ],
        out_specs=pl.BlockSpec((1, _RB, N), lambda c, p, i: (c, p * i, 0)),
        out_shape=jax.ShapeDtypeStruct((2, N, N), jnp.bfloat16),
        scratch_shapes=[
            pltpu.VMEM((N, N), jnp.bfloat16),
            pltpu.VMEM((N, N), jnp.bfloat16),
            pltpu.VMEM((1, N), jnp.float32),
            pltpu.VMEM((1, N), jnp.float32),
        ],
        compiler_params=pltpu.CompilerParams(
            dimension_semantics=("arbitrary", "arbitrary", "arbitrary")),
    )(F, A)

    x = seqs.reshape(B * T, NB)
    ib2 = I_B.reshape(1, NB)
    xs_t = pl.pallas_call(
        _basket_body,
        grid=(B // _BB,),
        in_specs=[
            pl.BlockSpec((_XB, NB), lambda i: (i, 0)),
            pl.BlockSpec((2, N, N), lambda i: (0, 0, 0)),
            pl.BlockSpec((1, NB), lambda i: (0, 0)),
            pl.BlockSpec((2, EMBED, NB), lambda i: (0, 0, 0)),
            pl.BlockSpec((2, 1, EMBED), lambda i: (0, 0, 0)),
            pl.BlockSpec((EMBED, 2 * EMBED), lambda i: (0, 0)),
            pl.BlockSpec((1, EMBED), lambda i: (0, 0)),
        ],
        out_specs=pl.BlockSpec((T, _BB, EMBED), lambda i: (0, i, 0)),
        out_shape=jax.ShapeDtypeStruct((T, B, EMBED), jnp.float32),
        compiler_params=pltpu.CompilerParams(
            dimension_semantics=("arbitrary",)),
    )(x, H2, ib2, lin_W, lin_b.reshape(2, 1, EMBED), proj_W,
      proj_b.reshape(1, EMBED))

    out = pl.pallas_call(
        _lstm_body,
        in_specs=[
            pl.BlockSpec((T, B, EMBED), lambda: (0, 0, 0)),
            pl.BlockSpec((4 * RNN, EMBED), lambda: (0, 0)),
            pl.BlockSpec((4 * RNN, RNN), lambda: (0, 0)),
            pl.BlockSpec((1, 4 * RNN), lambda: (0, 0)),
            pl.BlockSpec((1, 4 * RNN), lambda: (0, 0)),
            pl.BlockSpec((B, RNN), lambda: (0, 0)),
            pl.BlockSpec((B, RNN), lambda: (0, 0)),
            pl.BlockSpec((B, 1), lambda: (0, 0)),
            pl.BlockSpec((NB, RNN), lambda: (0, 0)),
            pl.BlockSpec((1, NB), lambda: (0, 0)),
        ],
        out_specs=pl.BlockSpec((B, NB), lambda: (0, 0)),
        out_shape=jax.ShapeDtypeStruct((B, NB), jnp.float32),
        scratch_shapes=[pltpu.VMEM((T * B, 4 * RNN), jnp.float32)],
    )(xs_t, W_ih, W_hh, b_ih.reshape(1, 4 * RNN), b_hh.reshape(1, 4 * RNN),
      h0.reshape(B, RNN), c0.reshape(B, RNN),
      seq_len.astype(jnp.int32).reshape(B, 1), h2i_W, ib2)

    return out
